# baseline (speedup 1.0000x reference)
import functools

import jax
import jax.numpy as jnp
from jax import lax
from jax.experimental import pallas as pl
from jax.experimental.pallas import tpu as pltpu
from jax.experimental.pallas import tpu_sc as plsc

VOCAB = 100000
VOCABP = 100096
EMBED = 64
BATCH = 1024


def _make_sc_gather():
    info = plsc.get_sparse_core_info()
    nc, ns = info.num_cores, info.num_subcores
    nw = nc * ns
    b_per_w = BATCH // nw
    n_chunks = (b_per_w * EMBED) // 128

    mesh = plsc.VectorSubcoreMesh(core_axis_name="c", subcore_axis_name="s")

    @functools.partial(
        pl.kernel,
        mesh=mesh,
        out_type=jax.ShapeDtypeStruct((BATCH * EMBED // 128, 128), jnp.float32),
        scratch_types=[
            pltpu.VMEM((b_per_w,), jnp.int32),
            pltpu.VMEM((n_chunks, 128), jnp.int32),
            pltpu.VMEM((n_chunks, 128), jnp.float32),
            pltpu.SemaphoreType.DMA,
        ],
        compiler_params=pltpu.CompilerParams(use_tc_tiling_on_sc=False),
    )
    def gather_kernel(tflat_hbm, idx_hbm, out_hbm, idx_v, eidx_v, rows_v, sem):
        wid = lax.axis_index("s") * nc + lax.axis_index("c")
        base = wid * b_per_w
        pltpu.sync_copy(idx_hbm.at[pl.ds(base, b_per_w)], idx_v)
        lanes = lax.iota(jnp.int32, 16)
        for i in range(b_per_w):
            vi = idx_v[pl.ds((i // 16) * 16, 16)]
            xi = jnp.broadcast_to(vi[i % 16], (16,))
            for q in range(EMBED // 16):
                s = i * EMBED + 16 * q
                eidx_v[s // 128, pl.ds(s % 128, 16)] = (lanes + 16 * q) * VOCABP + xi
        copies = [
            pltpu.async_copy(tflat_hbm.at[eidx_v.at[j]], rows_v.at[j], sem)
            for j in range(n_chunks)
        ]
        for c in copies:
            c.wait()
        pltpu.sync_copy(rows_v, out_hbm.at[pl.ds(wid * n_chunks, n_chunks)])

    return gather_kernel


VB = 4096


def _proj_kernel(wt_ref, emb_ref, b_ref, out_ref):
    acc = lax.dot_general(
        wt_ref[...],
        emb_ref[...],
        dimension_numbers=(((0,), (1,)), ((), ())),
        preferred_element_type=jnp.float32,
    )
    ones = jnp.ones((1, BATCH), jnp.float32)
    bias = lax.dot_general(
        b_ref[...],
        ones,
        dimension_numbers=(((0,), (0,)), ((), ())),
        preferred_element_type=jnp.float32,
    )
    out_ref[...] = acc + bias


def _projection_t(emb, hidden_wt, hidden_b2d):
    grid = (VOCAB + VB - 1) // VB
    return pl.pallas_call(
        _proj_kernel,
        grid=(grid,),
        in_specs=[
            pl.BlockSpec((EMBED, VB), lambda i: (0, i)),
            pl.BlockSpec((BATCH, EMBED), lambda i: (0, 0)),
            pl.BlockSpec((1, VB), lambda i: (0, i)),
        ],
        out_specs=pl.BlockSpec((VB, BATCH), lambda i: (i, 0)),
        out_shape=jax.ShapeDtypeStruct((VOCAB, BATCH), jnp.float32),
    )(hidden_wt, emb, hidden_b2d)


@jax.jit
def kernel(x, embed_table, hidden_w, hidden_b):
    idx = x.astype(jnp.int32)
    tflat = jnp.pad(embed_table.T, ((0, 0), (0, VOCABP - VOCAB))).reshape(-1)
    emb = _make_sc_gather()(tflat, idx).reshape(BATCH, EMBED)
    out_t = _projection_t(emb, hidden_w.T, hidden_b.reshape(1, VOCAB))
    return out_t.T


# speedup vs baseline: 1.0104x; 1.0104x over previous
import functools

import jax
import jax.numpy as jnp
from jax import lax
from jax.experimental import pallas as pl
from jax.experimental.pallas import tpu as pltpu
from jax.experimental.pallas import tpu_sc as plsc

VOCAB = 100000
EMBED = 64
BATCH = 1024


def _make_sc_gather():
    info = plsc.get_sparse_core_info()
    nc, ns = info.num_cores, info.num_subcores
    nw = nc * ns
    b_per_w = BATCH // nw
    n_chunks = (b_per_w * EMBED) // 128

    mesh = plsc.VectorSubcoreMesh(core_axis_name="c", subcore_axis_name="s")

    @functools.partial(
        pl.kernel,
        mesh=mesh,
        out_type=jax.ShapeDtypeStruct((BATCH * EMBED // 128, 128), jnp.float32),
        scratch_types=[
            pltpu.VMEM((b_per_w,), jnp.int32),
            pltpu.VMEM((n_chunks, 128), jnp.int32),
            pltpu.VMEM((n_chunks, 128), jnp.float32),
            pltpu.SemaphoreType.DMA,
        ],
        compiler_params=pltpu.CompilerParams(use_tc_tiling_on_sc=False),
    )
    def gather_kernel(tflat_hbm, idx_hbm, out_hbm, idx_v, eidx_v, rows_v, sem):
        wid = lax.axis_index("s") * nc + lax.axis_index("c")
        base = wid * b_per_w
        pltpu.sync_copy(idx_hbm.at[pl.ds(base, b_per_w)], idx_v)
        lanes = lax.iota(jnp.int32, 16)
        for i in range(b_per_w):
            vi = idx_v[pl.ds((i // 16) * 16, 16)]
            xi = jnp.broadcast_to(vi[i % 16], (16,))
            for q in range(EMBED // 16):
                s = i * EMBED + 16 * q
                eidx_v[s // 128, pl.ds(s % 128, 16)] = (lanes + 16 * q) * VOCAB + xi
        copies = [
            pltpu.async_copy(tflat_hbm.at[eidx_v.at[j]], rows_v.at[j], sem)
            for j in range(n_chunks)
        ]
        for c in copies:
            c.wait()
        pltpu.sync_copy(rows_v, out_hbm.at[pl.ds(wid * n_chunks, n_chunks)])

    return gather_kernel


VB = 4096


def _proj_kernel(wt_ref, emb_ref, b_ref, out_ref):
    acc = lax.dot_general(
        wt_ref[...],
        emb_ref[...],
        dimension_numbers=(((0,), (1,)), ((), ())),
        preferred_element_type=jnp.float32,
    )
    ones = jnp.ones((1, BATCH), jnp.float32)
    bias = lax.dot_general(
        b_ref[...],
        ones,
        dimension_numbers=(((0,), (0,)), ((), ())),
        preferred_element_type=jnp.float32,
    )
    out_ref[...] = acc + bias


def _projection_t(emb, hidden_wt, hidden_b2d):
    grid = (VOCAB + VB - 1) // VB
    return pl.pallas_call(
        _proj_kernel,
        grid=(grid,),
        in_specs=[
            pl.BlockSpec((EMBED, VB), lambda i: (0, i)),
            pl.BlockSpec((BATCH, EMBED), lambda i: (0, 0)),
            pl.BlockSpec((1, VB), lambda i: (0, i)),
        ],
        out_specs=pl.BlockSpec((VB, BATCH), lambda i: (i, 0)),
        out_shape=jax.ShapeDtypeStruct((VOCAB, BATCH), jnp.float32),
    )(hidden_wt, emb, hidden_b2d)


@jax.jit
def kernel(x, embed_table, hidden_w, hidden_b):
    idx = x.astype(jnp.int32)
    tflat = embed_table.T.reshape(-1)
    emb = _make_sc_gather()(tflat, idx).reshape(BATCH, EMBED)
    out_t = _projection_t(emb, hidden_w.T, hidden_b.reshape(1, VOCAB))
    return out_t.T
